# baseline (speedup 1.0000x reference)
import jax
import jax.numpy as jnp
from jax import lax
from jax.experimental import pallas as pl
from jax.experimental.pallas import tpu as pltpu
from jax.experimental.pallas import tpu_sc as plsc

B, L = 4096, 200
VOCAB, D = 256, 128

_NUM_WORKERS = 32
_ROWS_PER_W = B // _NUM_WORKERS
_GROUPS = _ROWS_PER_W // 16


def _hist_body(ids_hbm, counts_hbm, ids_v, hist_v, in_sems, out_sems):
    nc = 2
    wid = lax.axis_index("s") * nc + lax.axis_index("c")
    base = wid * _ROWS_PER_W

    in_copy = pltpu.async_copy(
        ids_hbm.at[:, pl.ds(base, _ROWS_PER_W)], ids_v, in_sems[0]
    )

    zeros16 = jnp.zeros((16,), jnp.float32)
    iota16 = lax.iota(jnp.int32, 16)
    ones16 = jnp.ones((16,), jnp.float32)

    @plsc.parallel_loop(0, _ROWS_PER_W * VOCAB, step=16, unroll=8)
    def _zero(j):
        hist_v[pl.ds(j, 16)] = zeros16

    in_copy.wait()
    out_copies = []
    for g in range(_GROUPS):
        hist_off = (g * 16 + iota16) * VOCAB

        @plsc.parallel_loop(0, L, unroll=8)
        def _tok(l):
            ids16 = ids_v[l, pl.ds(g * 16, 16)]
            plsc.addupdate_scatter(hist_v, [hist_off + ids16], ones16)

        out_copies.append(
            pltpu.async_copy(
                hist_v.at[pl.ds(g * 16 * VOCAB, 16 * VOCAB)],
                counts_hbm.at[pl.ds((base + g * 16) * VOCAB, 16 * VOCAB)],
                out_sems[g],
            )
        )
    for c in out_copies:
        c.wait()


_hist_call = pl.kernel(
    _hist_body,
    out_type=jax.ShapeDtypeStruct((B * VOCAB,), jnp.float32),
    mesh=plsc.VectorSubcoreMesh(core_axis_name="c", subcore_axis_name="s"),
    scratch_types=[
        pltpu.VMEM((L, _ROWS_PER_W), jnp.int32),
        pltpu.VMEM((_ROWS_PER_W * VOCAB,), jnp.float32),
        [pltpu.SemaphoreType.DMA],
        [pltpu.SemaphoreType.DMA] * _GROUPS,
    ],
    compiler_params=pltpu.CompilerParams(needs_layout_passes=False),
)


def _mm_body(counts_ref, emb_ref, out_ref):
    counts = counts_ref[...].reshape(_BB, VOCAB)
    acc = jax.lax.dot_general(
        counts, emb_ref[...],
        (((1,), (0,)), ((), ())),
        preferred_element_type=jnp.float32,
    )
    out_ref[...] = acc * (1.0 / L)


_BB = 512

_mm_call = pl.pallas_call(
    _mm_body,
    grid=(B // _BB,),
    in_specs=[
        pl.BlockSpec((_BB * VOCAB,), lambda i: (i,)),
        pl.BlockSpec((VOCAB, D), lambda i: (0, 0)),
    ],
    out_specs=pl.BlockSpec((_BB, D), lambda i: (i, 0)),
    out_shape=jax.ShapeDtypeStruct((B, D), jnp.float32),
)


@jax.jit
def kernel(token_ids, embedding):
    ids_t = token_ids.astype(jnp.int32).T
    counts = _hist_call(ids_t)
    counts = pltpu.with_memory_space_constraint(counts, pltpu.MemorySpace.HBM)
    return _mm_call(counts, embedding)


# speedup vs baseline: 1.1234x; 1.1234x over previous
import jax
import jax.numpy as jnp
from jax import lax
from jax.experimental import pallas as pl
from jax.experimental.pallas import tpu as pltpu
from jax.experimental.pallas import tpu_sc as plsc

B, L = 4096, 200
VOCAB, D = 256, 128

_NUM_WORKERS = 32
_ROWS_PER_W = B // _NUM_WORKERS
_GROUPS = _ROWS_PER_W // 16


def _hist_body(ids_hbm, counts_hbm, ids_v, hist_v, in_sems, out_sems):
    nc = 2
    wid = lax.axis_index("s") * nc + lax.axis_index("c")
    base = wid * _ROWS_PER_W

    in_copy = pltpu.async_copy(
        ids_hbm.at[:, pl.ds(base, _ROWS_PER_W)], ids_v, in_sems[0]
    )

    zeros16 = jnp.zeros((16,), jnp.float32)
    iota16 = lax.iota(jnp.int32, 16)
    ones16 = jnp.ones((16,), jnp.float32)

    @plsc.parallel_loop(0, _ROWS_PER_W * VOCAB, step=16, unroll=8)
    def _zero(j):
        hist_v[pl.ds(j, 16)] = zeros16

    in_copy.wait()
    out_copies = []
    for g in range(_GROUPS):
        hist_off = (g * 16 + iota16) * VOCAB

        @plsc.parallel_loop(0, L, unroll=8)
        def _tok(l):
            ids16 = ids_v[l, pl.ds(g * 16, 16)]
            plsc.addupdate_scatter(hist_v, [hist_off + ids16], ones16)

        out_copies.append(
            pltpu.async_copy(
                hist_v.at[pl.ds(g * 16 * VOCAB, 16 * VOCAB)],
                counts_hbm.at[pl.ds((base + g * 16) * VOCAB, 16 * VOCAB)],
                out_sems[g],
            )
        )
    for c in out_copies:
        c.wait()


_hist_call = pl.kernel(
    _hist_body,
    out_type=jax.ShapeDtypeStruct((B * VOCAB,), jnp.float32),
    mesh=plsc.VectorSubcoreMesh(core_axis_name="c", subcore_axis_name="s"),
    scratch_types=[
        pltpu.VMEM((L, _ROWS_PER_W), jnp.int32),
        pltpu.VMEM((_ROWS_PER_W * VOCAB,), jnp.float32),
        [pltpu.SemaphoreType.DMA],
        [pltpu.SemaphoreType.DMA] * _GROUPS,
    ],
    compiler_params=pltpu.CompilerParams(needs_layout_passes=False),
)


def _mm_body(counts_ref, emb_ref, out_ref):
    counts = counts_ref[...].reshape(_BB, VOCAB)
    acc = jax.lax.dot_general(
        counts, emb_ref[...],
        (((1,), (0,)), ((), ())),
        preferred_element_type=jnp.float32,
    )
    out_ref[...] = acc * (1.0 / L)


_BB = 2048

_mm_call = pl.pallas_call(
    _mm_body,
    grid=(B // _BB,),
    in_specs=[
        pl.BlockSpec((_BB * VOCAB,), lambda i: (i,)),
        pl.BlockSpec((VOCAB, D), lambda i: (0, 0)),
    ],
    out_specs=pl.BlockSpec((_BB, D), lambda i: (i, 0)),
    out_shape=jax.ShapeDtypeStruct((B, D), jnp.float32),
)


@jax.jit
def kernel(token_ids, embedding):
    ids_t = token_ids.astype(jnp.int32).T
    counts = _hist_call(ids_t)
    counts = pltpu.with_memory_space_constraint(counts, pltpu.MemorySpace.HBM)
    return _mm_call(counts, embedding)
